# baseline (speedup 1.0000x reference)
import functools

import jax
import jax.numpy as jnp
from jax import lax
from jax.experimental import pallas as pl
from jax.experimental.pallas import tpu as pltpu
from jax.experimental.pallas import tpu_sc as plsc

N = 10000
E = 320000
D = 128
ED = 16
H = 128
NG = 64
OUT = 10

_NC = 2
_NS = 16
_NW = _NC * _NS
_L = 16
_C = 80
_NCHUNK = E // _C
_ITERS = _NCHUNK // _NW
_NB = 2
_NBI = 4
_RC = 80
_NZCH = N // _RC


def _mm_body(a_ref, w_ref, o_ref):
    o_ref[...] = jnp.dot(a_ref[...], w_ref[...],
                         preferred_element_type=jnp.float32)


def _mm(a, w, blk):
    m, k = a.shape
    n = w.shape[1]
    grid = m // blk
    return pl.pallas_call(
        _mm_body,
        grid=(grid,),
        in_specs=[
            pl.BlockSpec((blk, k), lambda i: (i, 0)),
            pl.BlockSpec((k, n), lambda i: (0, 0)),
        ],
        out_specs=pl.BlockSpec((blk, n), lambda i: (i, 0)),
        out_shape=jax.ShapeDtypeStruct((m, n), jnp.float32),
    )(a, w)


@functools.lru_cache(maxsize=None)
def _sc_edge_kernel():
    return functools.partial(
        pl.kernel,
        out_type=jax.ShapeDtypeStruct((_NC, N, H), jnp.float32),
        mesh=plsc.VectorSubcoreMesh(core_axis_name="c", subcore_axis_name="s",
                                    num_cores=_NC, num_subcores=_NS),
        scratch_types=[
            pltpu.VMEM((_NBI, 2, _C), jnp.int32),
            pltpu.VMEM((_NB, _C, H), jnp.float32),
            pltpu.VMEM((_NB, _C, H), jnp.float32),
            pltpu.VMEM_SHARED((N, H), jnp.float32),
            pltpu.SemaphoreType.DMA,
            pltpu.SemaphoreType.DMA,
            pltpu.SemaphoreType.DMA,
            pltpu.SemaphoreType.DMA,
        ],
    )(_sc_edge_body)


def _sc_edge(hs, e, ei):
    return _sc_edge_kernel()(hs, e, ei.reshape(-1))


def _sc_edge_body(hs_hbm, e_hbm, ei_hbm, out_hbm, idx, ebuf, rows, agg,
                  sem_i, sem_e, sem_g, sem_s):
    c = lax.axis_index("c")
    s = lax.axis_index("s")
    wid = s * _NC + c

    def issue_idx(i, b):
        base = (i * _NW + wid) * _C
        pltpu.async_copy(ei_hbm.at[pl.ds(base, _C)], idx.at[b, 0], sem_i)
        pltpu.async_copy(ei_hbm.at[pl.ds(E + base, _C)], idx.at[b, 1], sem_i)

    def wait_idx(b):
        pltpu.make_async_copy(ei_hbm.at[pl.ds(0, _C)], idx.at[b, 0],
                              sem_i).wait()
        pltpu.make_async_copy(ei_hbm.at[pl.ds(0, _C)], idx.at[b, 1],
                              sem_i).wait()

    def issue_e(i, b):
        base = (i * _NW + wid) * _C
        pltpu.async_copy(e_hbm.at[pl.ds(base, _C), :], ebuf.at[b], sem_e)

    def wait_e(b):
        pltpu.make_async_copy(e_hbm.at[pl.ds(0, _C), :], ebuf.at[b],
                              sem_e).wait()

    def issue_gather(b, bi):
        pltpu.async_copy(hs_hbm.at[idx.at[bi, 0]], rows.at[b], sem_g)

    def wait_gather(b, bi):
        pltpu.make_async_copy(hs_hbm.at[idx.at[bi, 0]], rows.at[b],
                              sem_g).wait()

    def issue_scatter(b, bi):
        pltpu.async_copy(rows.at[b], agg.at[idx.at[bi, 1]], sem_s, add=True)

    def wait_scatter(b, bi):
        pltpu.make_async_copy(rows.at[b], agg.at[idx.at[bi, 1]],
                              sem_s).wait()

    zv = jnp.zeros((_L,), jnp.float32)

    @pl.loop(0, _RC)
    def _zero_rows(r):
        for j in range(H // _L):
            rows[0, r, pl.ds(j * _L, _L)] = zv

    for j in range((_NZCH + _NS - 1) // _NS):
        t = j * _NS + s

        @pl.when(t < _NZCH)
        def _():
            pltpu.sync_copy(rows.at[0], agg.at[pl.ds(t * _RC, _RC)])
    plsc.subcore_barrier()

    issue_idx(0, 0)
    issue_idx(1, 1)
    issue_e(0, 0)
    wait_idx(0)
    issue_gather(0, 0)

    @pl.loop(0, _ITERS)
    def _pipe(i):
        b = lax.rem(i, _NB)
        bn = lax.rem(i + 1, _NB)
        bi = lax.rem(i, _NBI)
        bin_ = lax.rem(i + 1, _NBI)

        @pl.when(i >= 1)
        def _():
            wait_scatter(bn, lax.rem(i + 3, _NBI))

        @pl.when(i + 1 < _ITERS)
        def _():
            wait_idx(bin_)
            issue_gather(bn, bin_)

        @pl.when(i + 2 < _ITERS)
        def _():
            issue_idx(i + 2, lax.rem(i + 2, _NBI))

        @pl.when(i + 1 < _ITERS)
        def _():
            issue_e(i + 1, bn)

        wait_gather(b, bi)
        wait_e(b)

        @pl.loop(0, _C)
        def _relu_rows(r):
            for j in range(H // _L):
                sl = pl.ds(j * _L, _L)
                rows[b, r, sl] = jnp.maximum(rows[b, r, sl] + ebuf[b, r, sl],
                                             0.0)

        issue_scatter(b, bi)

    wait_scatter(0, 0)

    plsc.subcore_barrier()
    for j in range((_NZCH + _NS - 1) // _NS):
        t = j * _NS + s

        @pl.when(t < _NZCH)
        def _():
            pltpu.sync_copy(agg.at[pl.ds(t * _RC, _RC)],
                            out_hbm.at[c, pl.ds(t * _RC, _RC)])


def _finish_body(hs_ref, agg0_ref, agg1_ref, batch_ref, eps_ref, wo_ref,
                 bo_ref, wxn_ref, hsn_ref, pooled_ref):
    i = pl.program_id(0)
    t = (1.0 + eps_ref[0, 0]) * hs_ref[...] + agg0_ref[...] + agg1_ref[...]
    u = jnp.dot(t, wo_ref[...], preferred_element_type=jnp.float32)
    h = jnp.maximum(u + bo_ref[...], 0.0)
    hsn_ref[...] = jnp.dot(h, wxn_ref[...], preferred_element_type=jnp.float32)
    oh = (batch_ref[...] ==
          lax.broadcasted_iota(jnp.int32, (1, NG), 1)).astype(jnp.float32)
    contrib = lax.dot_general(oh, h, (((0,), (0,)), ((), ())),
                              preferred_element_type=jnp.float32)

    @pl.when(i == 0)
    def _():
        pooled_ref[...] = jnp.zeros_like(pooled_ref)

    pooled_ref[...] += contrib


def _finish(hs, agg0, agg1, batch2, eps, wo, bo2, wxn):
    blk = 1000
    grid = N // blk
    return pl.pallas_call(
        _finish_body,
        grid=(grid,),
        in_specs=[
            pl.BlockSpec((blk, H), lambda i: (i, 0)),
            pl.BlockSpec((blk, H), lambda i: (i, 0)),
            pl.BlockSpec((blk, H), lambda i: (i, 0)),
            pl.BlockSpec((blk, 1), lambda i: (i, 0)),
            pl.BlockSpec((1, 1), lambda i: (0, 0)),
            pl.BlockSpec((H, H), lambda i: (0, 0)),
            pl.BlockSpec((1, H), lambda i: (0, 0)),
            pl.BlockSpec((H, H), lambda i: (0, 0)),
        ],
        out_specs=[
            pl.BlockSpec((blk, H), lambda i: (i, 0)),
            pl.BlockSpec((NG, H), lambda i: (0, 0)),
        ],
        out_shape=[
            jax.ShapeDtypeStruct((N, H), jnp.float32),
            jax.ShapeDtypeStruct((NG, H), jnp.float32),
        ],
    )(hs, agg0, agg1, batch2, eps, wo, bo2, wxn)


def _head_body(p1_ref, p2_ref, p3_ref, batch_ref, w1_ref, b1_ref, w2_ref,
               b2_ref, o_ref):
    oh = (batch_ref[...] ==
          lax.broadcasted_iota(jnp.int32, (1, NG), 1)).astype(jnp.float32)
    ones = jnp.ones((N, 1), jnp.float32)
    cnt = lax.dot_general(oh, ones, (((0,), (0,)), ((), ())),
                          preferred_element_type=jnp.float32)
    denom = jnp.maximum(cnt, 1.0)
    hcat = jnp.concatenate(
        [p1_ref[...] / denom, p2_ref[...] / denom, p3_ref[...] / denom],
        axis=1)
    hl = jnp.maximum(
        jnp.dot(hcat, w1_ref[...], preferred_element_type=jnp.float32)
        + b1_ref[...], 0.0)
    logits = jnp.dot(hl, w2_ref[...], preferred_element_type=jnp.float32) \
        + b2_ref[...]
    m = jnp.max(logits, axis=1, keepdims=True)
    lse = jnp.log(jnp.sum(jnp.exp(logits - m), axis=1, keepdims=True)) + m
    o_ref[...] = logits - lse


def _head(p1, p2, p3, batch2, w1, b12, w2, b22):
    return pl.pallas_call(
        _head_body,
        out_shape=jax.ShapeDtypeStruct((NG, OUT), jnp.float32),
    )(p1, p2, p3, batch2, w1, b12, w2, b22)


def kernel(x, edge_index, edge_attr, batch, Wx1, We1, Wo1, bo1, eps1, Wx2,
           We2, Wo2, bo2, eps2, Wx3, We3, Wo3, bo3, eps3, lin1_W, lin1_b,
           lin2_W, lin2_b):
    batch2 = batch.reshape(N, 1)
    eye = jnp.eye(H, dtype=jnp.float32)

    hs = _mm(x, Wx1, 1000)
    pooled = []
    layers = [
        (We1, Wo1, bo1, eps1, Wx2),
        (We2, Wo2, bo2, eps2, Wx3),
        (We3, Wo3, bo3, eps3, eye),
    ]
    for we, wo, bo, eps, wxn in layers:
        e = _mm(edge_attr, we, 4000)
        aggp = _sc_edge(hs, e, edge_index)
        hs, p = _finish(hs, aggp[0], aggp[1], batch2, eps.reshape(1, 1), wo,
                        bo.reshape(1, H), wxn)
        pooled.append(p)

    return _head(pooled[0], pooled[1], pooled[2], batch2, lin1_W,
                 lin1_b.reshape(1, 3 * H), lin2_W, lin2_b.reshape(1, OUT))


# speedup vs baseline: 1.3946x; 1.3946x over previous
import functools

import jax
import jax.numpy as jnp
from jax import lax
from jax.experimental import pallas as pl
from jax.experimental.pallas import tpu as pltpu
from jax.experimental.pallas import tpu_sc as plsc

N = 10000
E = 320000
D = 128
ED = 16
H = 128
NG = 64
OUT = 10

_NC = 2
_NS = 16
_NW = _NC * _NS
_L = 16
_C = 128
_NCHUNK = E // _C
_NIT = (_NCHUNK + _NW - 1) // _NW
_NB = 2
_NBI = 4
_RC = 80
_NZCH = N // _RC


def _mm_body(a_ref, w_ref, o_ref):
    o_ref[...] = jnp.dot(a_ref[...], w_ref[...],
                         preferred_element_type=jnp.float32)


def _mm(a, w, blk):
    m, k = a.shape
    n = w.shape[1]
    return pl.pallas_call(
        _mm_body,
        grid=(m // blk,),
        in_specs=[
            pl.BlockSpec((blk, k), lambda i: (i, 0)),
            pl.BlockSpec((k, n), lambda i: (0, 0)),
        ],
        out_specs=pl.BlockSpec((blk, n), lambda i: (i, 0)),
        out_shape=jax.ShapeDtypeStruct((m, n), jnp.float32),
    )(a, w)


def _pack_rows(h):
    b = lax.bitcast_convert_type(h, jnp.uint32)
    rb = b + jnp.uint32(0x7FFF) + ((b >> 16) & jnp.uint32(1))
    lo = jnp.concatenate([rb[:, 0:16], rb[:, 32:48], rb[:, 64:80],
                          rb[:, 96:112]], axis=1)
    hi = jnp.concatenate([rb[:, 16:32], rb[:, 48:64], rb[:, 80:96],
                          rb[:, 112:128]], axis=1)
    return lax.bitcast_convert_type((lo >> 16) | (hi & jnp.uint32(0xFFFF0000)),
                                    jnp.int32)


def _mm_e_body(a_ref, w_ref, op_ref):
    e = jnp.dot(a_ref[...], w_ref[...], preferred_element_type=jnp.float32)
    pk = _pack_rows(e)
    half = pk.shape[0] // 2
    op_ref[...] = jnp.concatenate([pk[:half], pk[half:]], axis=1)


def _mm_e(a, w, blk):
    m, k = a.shape
    n = w.shape[1]
    return pl.pallas_call(
        _mm_e_body,
        grid=(m // blk,),
        in_specs=[
            pl.BlockSpec((blk, k), lambda i: (i, 0)),
            pl.BlockSpec((k, n), lambda i: (0, 0)),
        ],
        out_specs=pl.BlockSpec((blk // 2, n), lambda i: (i, 0)),
        out_shape=jax.ShapeDtypeStruct((m // 2, n), jnp.int32),
    )(a, w)


@functools.lru_cache(maxsize=None)
def _sc_edge_kernel():
    return functools.partial(
        pl.kernel,
        out_type=jax.ShapeDtypeStruct((_NC, N, H), jnp.float32),
        mesh=plsc.VectorSubcoreMesh(core_axis_name="c", subcore_axis_name="s",
                                    num_cores=_NC, num_subcores=_NS),
        scratch_types=[
            pltpu.VMEM((_NBI, 2, _C), jnp.int32),
            pltpu.VMEM((_NB, _C // 2, H), jnp.int32),
            pltpu.VMEM((_NB, _C, H), jnp.float32),
            pltpu.VMEM_SHARED((N, H), jnp.float32),
            pltpu.SemaphoreType.DMA,
            pltpu.SemaphoreType.DMA,
            pltpu.SemaphoreType.DMA,
            pltpu.SemaphoreType.DMA,
        ],
    )(_sc_edge_body)


def _sc_edge(hs, ep, ei3):
    return _sc_edge_kernel()(hs, ep, ei3)


def _sc_edge_body(hs_hbm, ep_hbm, ei_hbm, out_hbm, idx, ebuf, rows, agg,
                  sem_i, sem_e, sem_g, sem_s):
    c = lax.axis_index("c")
    s = lax.axis_index("s")
    wid = s * _NC + c

    def chunk_of(i):
        return i * _NW + wid

    def valid(i):
        return chunk_of(i) < _NCHUNK

    def issue_idx(i, b):
        pltpu.async_copy(ei_hbm.at[chunk_of(i)], idx.at[b], sem_i)

    def wait_idx(b):
        pltpu.make_async_copy(ei_hbm.at[0], idx.at[b], sem_i).wait()

    def issue_e(i, b):
        base = chunk_of(i) * (_C // 2)
        pltpu.async_copy(ep_hbm.at[pl.ds(base, _C // 2), :], ebuf.at[b],
                         sem_e)

    def wait_e(b):
        pltpu.make_async_copy(ep_hbm.at[pl.ds(0, _C // 2), :], ebuf.at[b],
                              sem_e).wait()

    def issue_gather(b, bi):
        pltpu.async_copy(hs_hbm.at[idx.at[bi, 0]], rows.at[b], sem_g)

    def wait_gather(b, bi):
        pltpu.make_async_copy(hs_hbm.at[idx.at[bi, 0]], rows.at[b],
                              sem_g).wait()

    def issue_scatter(b, bi):
        pltpu.async_copy(rows.at[b], agg.at[idx.at[bi, 1]], sem_s, add=True)

    def wait_scatter(b, bi):
        pltpu.make_async_copy(rows.at[b], agg.at[idx.at[bi, 1]],
                              sem_s).wait()

    zv = jnp.zeros((_L,), jnp.float32)

    @pl.loop(0, _RC)
    def _zero_rows(r):
        for j in range(H // _L):
            rows[0, r, pl.ds(j * _L, _L)] = zv

    for j in range((_NZCH + _NS - 1) // _NS):
        t = j * _NS + s

        @pl.when(t < _NZCH)
        def _():
            pltpu.sync_copy(rows.at[0, pl.ds(0, _RC)],
                            agg.at[pl.ds(t * _RC, _RC)])
    plsc.subcore_barrier()

    issue_idx(0, 0)
    issue_idx(1, 1)
    issue_e(0, 0)
    wait_idx(0)
    issue_gather(0, 0)

    @pl.loop(0, _NIT)
    def _pipe(i):
        b = lax.rem(i, _NB)
        bn = lax.rem(i + 1, _NB)
        bi = lax.rem(i, _NBI)
        bin_ = lax.rem(i + 1, _NBI)

        @pl.when(jnp.logical_and(i >= 1, valid(i - 1)))
        def _():
            wait_scatter(bn, lax.rem(i + 3, _NBI))

        @pl.when(valid(i + 1))
        def _():
            wait_idx(bin_)
            issue_gather(bn, bin_)

        @pl.when(valid(i + 2))
        def _():
            issue_idx(i + 2, lax.rem(i + 2, _NBI))

        @pl.when(valid(i + 1))
        def _():
            issue_e(i + 1, bn)

        @pl.when(valid(i))
        def _():
            wait_gather(b, bi)
            wait_e(b)

            @pl.loop(0, _C // 2)
            def _relu_rows(rr):
                for k in range(2):
                    r = 2 * rr + k
                    for g in range(H // (2 * _L)):
                        w = ebuf[b, rr, pl.ds(64 * k + g * _L, _L)]
                        lo = lax.bitcast_convert_type(w << 16, jnp.float32)
                        hi = lax.bitcast_convert_type(
                            w & jnp.int32(-65536), jnp.float32)
                        sl_lo = pl.ds(2 * g * _L, _L)
                        sl_hi = pl.ds((2 * g + 1) * _L, _L)
                        rows[b, r, sl_lo] = jnp.maximum(
                            rows[b, r, sl_lo] + lo, 0.0)
                        rows[b, r, sl_hi] = jnp.maximum(
                            rows[b, r, sl_hi] + hi, 0.0)

            issue_scatter(b, bi)

    @pl.when(valid(_NIT - 1))
    def _():
        wait_scatter(0, 0)

    plsc.subcore_barrier()
    for j in range((_NZCH + _NS - 1) // _NS):
        t = j * _NS + s

        @pl.when(t < _NZCH)
        def _():
            pltpu.sync_copy(agg.at[pl.ds(t * _RC, _RC)],
                            out_hbm.at[c, pl.ds(t * _RC, _RC)])


def _finish_body(hs_ref, agg0_ref, agg1_ref, batch_ref, eps_ref, wo_ref,
                 bo_ref, wxn_ref, hsn_ref, pooled_ref):
    i = pl.program_id(0)
    t = (1.0 + eps_ref[0, 0]) * hs_ref[...] + agg0_ref[...] + agg1_ref[...]
    u = jnp.dot(t, wo_ref[...], preferred_element_type=jnp.float32)
    h = jnp.maximum(u + bo_ref[...], 0.0)
    hsn_ref[...] = jnp.dot(h, wxn_ref[...], preferred_element_type=jnp.float32)
    oh = (batch_ref[...] ==
          lax.broadcasted_iota(jnp.int32, (1, NG), 1)).astype(jnp.float32)
    contrib = lax.dot_general(oh, h, (((0,), (0,)), ((), ())),
                              preferred_element_type=jnp.float32)

    @pl.when(i == 0)
    def _():
        pooled_ref[...] = jnp.zeros_like(pooled_ref)

    pooled_ref[...] += contrib


def _finish(hs, agg0, agg1, batch2, eps, wo, bo2, wxn):
    blk = 1000
    return pl.pallas_call(
        _finish_body,
        grid=(N // blk,),
        in_specs=[
            pl.BlockSpec((blk, H), lambda i: (i, 0)),
            pl.BlockSpec((blk, H), lambda i: (i, 0)),
            pl.BlockSpec((blk, H), lambda i: (i, 0)),
            pl.BlockSpec((blk, 1), lambda i: (i, 0)),
            pl.BlockSpec((1, 1), lambda i: (0, 0)),
            pl.BlockSpec((H, H), lambda i: (0, 0)),
            pl.BlockSpec((1, H), lambda i: (0, 0)),
            pl.BlockSpec((H, H), lambda i: (0, 0)),
        ],
        out_specs=[
            pl.BlockSpec((blk, H), lambda i: (i, 0)),
            pl.BlockSpec((NG, H), lambda i: (0, 0)),
        ],
        out_shape=[
            jax.ShapeDtypeStruct((N, H), jnp.float32),
            jax.ShapeDtypeStruct((NG, H), jnp.float32),
        ],
    )(hs, agg0, agg1, batch2, eps, wo, bo2, wxn)


def _head_body(p1_ref, p2_ref, p3_ref, batch_ref, w1_ref, b1_ref, w2_ref,
               b2_ref, o_ref):
    oh = (batch_ref[...] ==
          lax.broadcasted_iota(jnp.int32, (1, NG), 1)).astype(jnp.float32)
    ones = jnp.ones((N, 1), jnp.float32)
    cnt = lax.dot_general(oh, ones, (((0,), (0,)), ((), ())),
                          preferred_element_type=jnp.float32)
    denom = jnp.maximum(cnt, 1.0)
    hcat = jnp.concatenate(
        [p1_ref[...] / denom, p2_ref[...] / denom, p3_ref[...] / denom],
        axis=1)
    hl = jnp.maximum(
        jnp.dot(hcat, w1_ref[...], preferred_element_type=jnp.float32)
        + b1_ref[...], 0.0)
    logits = jnp.dot(hl, w2_ref[...], preferred_element_type=jnp.float32) \
        + b2_ref[...]
    m = jnp.max(logits, axis=1, keepdims=True)
    lse = jnp.log(jnp.sum(jnp.exp(logits - m), axis=1, keepdims=True)) + m
    o_ref[...] = logits - lse


def _head(p1, p2, p3, batch2, w1, b12, w2, b22):
    return pl.pallas_call(
        _head_body,
        out_shape=jax.ShapeDtypeStruct((NG, OUT), jnp.float32),
    )(p1, p2, p3, batch2, w1, b12, w2, b22)


def kernel(x, edge_index, edge_attr, batch, Wx1, We1, Wo1, bo1, eps1, Wx2,
           We2, Wo2, bo2, eps2, Wx3, We3, Wo3, bo3, eps3, lin1_W, lin1_b,
           lin2_W, lin2_b):
    batch2 = batch.reshape(N, 1)
    eye = jnp.eye(H, dtype=jnp.float32)
    ei3 = edge_index.reshape(2, _NCHUNK, _C).transpose(1, 0, 2)
    ea_shuf = edge_attr.reshape(E // 4000, 2000, 2, ED).transpose(
        0, 2, 1, 3).reshape(E, ED)

    hs = _mm(x, Wx1, 1000)
    pooled = []
    layers = [
        (We1, Wo1, bo1, eps1, Wx2),
        (We2, Wo2, bo2, eps2, Wx3),
        (We3, Wo3, bo3, eps3, eye),
    ]
    for we, wo, bo, eps, wxn in layers:
        ep = _mm_e(ea_shuf, we, 4000)
        aggp = _sc_edge(hs, ep, ei3)
        hs, p = _finish(hs, aggp[0], aggp[1], batch2, eps.reshape(1, 1), wo,
                        bo.reshape(1, H), wxn)
        pooled.append(p)

    return _head(pooled[0], pooled[1], pooled[2], batch2, lin1_W,
                 lin1_b.reshape(1, 3 * H), lin2_W, lin2_b.reshape(1, OUT))
